# baseline (speedup 1.0000x reference)
import jax
import jax.numpy as jnp
from jax import lax
from jax.experimental import pallas as pl
from jax.experimental.pallas import tpu as pltpu
from jax.experimental.pallas import tpu_sc as plsc

NC = 2
NS = 16
LN = 16
K = 128
CW = 24
PAD = 8

_INTERPRET = False


def _encode_body(a_ref, w_ref, al_ref, o_ref):
    a = a_ref[...]
    p = jnp.where(a >= 0.0, a, al_ref[0, 0] * a)
    o_ref[...] = lax.dot_general(
        p, w_ref[...], (((1,), (1,)), ((), ())),
        preferred_element_type=jnp.float32)


def _encode(node_attr, W_enc, prelu_a):
    N, IN = node_attr.shape
    HF = IN // 2
    BN = 1000
    NB = N // BN
    return pl.pallas_call(
        _encode_body,
        grid=(2, NB),
        in_specs=[
            pl.BlockSpec((BN, IN), lambda h, i: (i, 0)),
            pl.BlockSpec((HF, IN), lambda h, i: (h, 0)),
            pl.BlockSpec(memory_space=pltpu.SMEM),
        ],
        out_specs=pl.BlockSpec((BN, HF), lambda h, i: (h * NB + i, 0)),
        out_shape=jax.ShapeDtypeStruct((2 * N, HF), jnp.float32),
        interpret=_INTERPRET,
    )(node_attr, W_enc, prelu_a.reshape(1, 1))


def _sc_aggregate(x_flat, packed, zeros_hbm):
    twoN, HF = x_flat.shape
    N = twoN // 2
    NCHUNK = packed.shape[0]
    CH = NCHUNK // NS
    NPAIR = CH // 2
    NR = N + PAD
    RPT = ((N // NS + 7) // 8) * 8
    RPT_L = N - (NS - 1) * RPT
    NW = N * CW
    NWP = NW + 128
    CPW = ((NW // NS + 127) // 128) * 128
    CPW_L = NW - (NS - 1) * CPW

    mesh = plsc.VectorSubcoreMesh(
        core_axis_name="c", subcore_axis_name="s",
        num_cores=NC, num_subcores=NS)

    def body(x_hbm, idx_hbm, zero_hbm, acc_out, cnt_out,
             ib0, ib1, rows0, rows1, ones_v,
             acc_sh, cnt_sh, gsem, ssem0, ssem1, csem):
        cid = lax.axis_index("c")
        wid = lax.axis_index("s")
        coff = cid * N

        def init_wb(rows, words, store):
            r0 = pl.multiple_of(wid * RPT, 8)
            w0 = pl.multiple_of(wid * CPW, 128)
            if store:
                pltpu.sync_copy(acc_sh.at[pl.ds(r0, rows)],
                                acc_out.at[pl.ds(pl.multiple_of(
                                    coff + wid * RPT, 8), rows)])
                pltpu.sync_copy(
                    cnt_sh.at[pl.ds(w0, words)],
                    cnt_out.at[pl.ds(pl.multiple_of(
                        cid * NW + wid * CPW, 128), words)])
            else:
                pltpu.sync_copy(x_hbm.at[pl.ds(pl.multiple_of(
                                    coff + wid * RPT, 8), rows)],
                                acc_sh.at[pl.ds(r0, rows)])
                pltpu.sync_copy(zero_hbm.at[pl.ds(w0, words)],
                                cnt_sh.at[pl.ds(w0, words)])

        @pl.when(wid < NS - 1)
        def _():
            init_wb(RPT, CPW, False)

        @pl.when(wid == NS - 1)
        def _():
            init_wb(RPT_L, CPW_L, False)
            pltpu.sync_copy(zero_hbm.at[pl.ds(NW, NWP - NW)],
                            cnt_sh.at[pl.ds(NW, NWP - NW)])

        for j in range(K // LN):
            ones_v[pl.ds(j * LN, LN)] = jnp.full((LN,), 1.0, jnp.float32)

        plsc.subcore_barrier()

        cbase = wid * CH

        def stage(chunk_idx, ib):
            pltpu.sync_copy(idx_hbm.at[cbase + chunk_idx], ib)

        def wait_scatter(rv, ib, sem):
            pltpu.make_async_copy(rv, acc_sh.at[ib.at[2]], sem).wait()

        def wait_cnt(ib):
            pltpu.make_async_copy(ones_v, cnt_sh.at[ib.at[3]], csem).wait()

        stage(0, ib0)

        def pair(j, carry):
            c0 = 2 * j
            g0 = pltpu.async_copy(x_hbm.at[ib0.at[cid]], rows0, gsem)

            @pl.when(j > 0)
            def _():
                wait_scatter(rows1, ib1, ssem1)

                @pl.when(cid == 1)
                def _():
                    wait_cnt(ib1)
            stage(c0 + 1, ib1)
            g1 = pltpu.async_copy(x_hbm.at[ib1.at[cid]], rows1, gsem)
            g0.wait()
            pltpu.async_copy(rows0, acc_sh.at[ib0.at[2]], ssem0, add=True)

            @pl.when(cid == 0)
            def _():
                pltpu.async_copy(ones_v, cnt_sh.at[ib0.at[3]], csem, add=True)

            @pl.when(j < NPAIR - 1)
            def _():
                wait_scatter(rows0, ib0, ssem0)

                @pl.when(cid == 0)
                def _():
                    wait_cnt(ib0)
                stage(c0 + 2, ib0)
            g1.wait()
            pltpu.async_copy(rows1, acc_sh.at[ib1.at[2]], ssem1, add=True)

            @pl.when(cid == 1)
            def _():
                pltpu.async_copy(ones_v, cnt_sh.at[ib1.at[3]], csem, add=True)
            return carry
        lax.fori_loop(0, NPAIR, pair, 0)

        wait_scatter(rows0, ib0, ssem0)
        wait_scatter(rows1, ib1, ssem1)
        wait_cnt(ib0)

        plsc.subcore_barrier()

        @pl.when(wid < NS - 1)
        def _():
            init_wb(RPT, CPW, True)

        @pl.when(wid == NS - 1)
        def _():
            init_wb(RPT_L, CPW_L, True)

    fn = pl.kernel(
        body,
        out_type=[
            jax.ShapeDtypeStruct((twoN, HF), jnp.float32),
            jax.ShapeDtypeStruct((NC * N * CW,), jnp.float32),
        ],
        mesh=mesh,
        scratch_types=[
            pltpu.VMEM((4, K), jnp.int32),
            pltpu.VMEM((4, K), jnp.int32),
            pltpu.VMEM((K, HF), jnp.float32),
            pltpu.VMEM((K, HF), jnp.float32),
            pltpu.VMEM((K,), jnp.float32),
            pltpu.VMEM_SHARED((NR, HF), jnp.float32),
            pltpu.VMEM_SHARED((NWP,), jnp.float32),
            pltpu.SemaphoreType.DMA,
            pltpu.SemaphoreType.DMA,
            pltpu.SemaphoreType.DMA,
            pltpu.SemaphoreType.DMA,
        ],
        interpret=_INTERPRET,
    )
    return fn(x_flat, packed, zeros_hbm)


def _mlp_body(a0_ref, a1_ref, c0_ref, c1_ref, t_ref, ca_ref, w1_ref, b1_ref,
              w2_ref, b2_ref, o_ref):
    aggr = jnp.concatenate([a0_ref[...], a1_ref[...]], axis=1)
    cnt = c0_ref[...] + c1_ref[...]
    aggr = aggr + lax.dot_general(
        cnt, t_ref[...], (((1,), (0,)), ((), ())),
        preferred_element_type=jnp.float32) + ca_ref[...]
    h = lax.dot_general(
        aggr, w1_ref[...], (((1,), (1,)), ((), ())),
        preferred_element_type=jnp.float32) + b1_ref[...]
    h = jnp.maximum(h, 0.0)
    o_ref[...] = lax.dot_general(
        h, w2_ref[...], (((1,), (1,)), ((), ())),
        preferred_element_type=jnp.float32) + b2_ref[...]


def _mlp(acc, cnt2, T_pad, const_attr, W1, b1, W2, b2):
    twoN, HF = acc.shape
    N = twoN // 2
    IN = 2 * HF
    HID = W1.shape[0]
    OUT = W2.shape[0]
    BN = 1000
    NB = N // BN
    return pl.pallas_call(
        _mlp_body,
        grid=(NB,),
        in_specs=[
            pl.BlockSpec((BN, HF), lambda i: (i, 0)),
            pl.BlockSpec((BN, HF), lambda i: (i + NB, 0)),
            pl.BlockSpec((BN, CW), lambda i: (i, 0)),
            pl.BlockSpec((BN, CW), lambda i: (i + NB, 0)),
            pl.BlockSpec((CW, IN), lambda i: (0, 0)),
            pl.BlockSpec((1, IN), lambda i: (0, 0)),
            pl.BlockSpec((HID, IN), lambda i: (0, 0)),
            pl.BlockSpec((1, HID), lambda i: (0, 0)),
            pl.BlockSpec((OUT, HID), lambda i: (0, 0)),
            pl.BlockSpec((1, OUT), lambda i: (0, 0)),
        ],
        out_specs=pl.BlockSpec((BN, OUT), lambda i: (i, 0)),
        out_shape=jax.ShapeDtypeStruct((N, OUT), jnp.float32),
        interpret=_INTERPRET,
    )(acc, acc, cnt2, cnt2, T_pad, const_attr, W1, b1, W2, b2)


def kernel(node_attr, edge_index, edge_type, edge_dire_type, W_enc, prelu_a,
           emb1, emb2, W1, b1, W2, b2):
    N, IN = node_attr.shape
    E = edge_index.shape[1]
    nd = emb2.shape[0]

    src = edge_index[0].astype(jnp.int32)
    dst = edge_index[1].astype(jnp.int32)
    code = edge_type.astype(jnp.int32) * nd + edge_dire_type.astype(jnp.int32)
    EPK = NS * K * 2
    EP = ((E + EPK - 1) // EPK) * EPK
    npad = EP - E
    src = jnp.concatenate([src, jnp.zeros((npad,), jnp.int32)])
    dst = jnp.concatenate([dst, jnp.full((npad,), N, jnp.int32)])
    code = jnp.concatenate([code, jnp.zeros((npad,), jnp.int32)])
    flat = dst * CW + code
    srcr = src.reshape(-1, K)
    packed = jnp.stack(
        [srcr, srcr + N, dst.reshape(-1, K), flat.reshape(-1, K)], axis=1)
    zeros = jnp.zeros((N * CW + 128,), jnp.float32)

    nb = emb1.shape[0]
    t_idx = jnp.repeat(jnp.arange(nb), nd)
    d_idx = jnp.tile(jnp.arange(nd), nb)
    T = emb1[t_idx] + emb2[d_idx]
    T_pad = jnp.zeros((CW, IN), jnp.float32).at[:nb * nd].set(T)
    const_attr = (emb1[4] + emb2[0]).reshape(1, IN)

    x_flat = _encode(node_attr, W_enc, prelu_a.astype(jnp.float32))
    acc, cnt = _sc_aggregate(x_flat, packed, zeros)
    cnt2 = cnt.reshape(2 * N, CW)
    return _mlp(acc, cnt2, T_pad, const_attr, W1, b1.reshape(1, -1),
                W2, b2.reshape(1, -1))


# speedup vs baseline: 1.4677x; 1.4677x over previous
import jax
import jax.numpy as jnp
from jax import lax
from jax.experimental import pallas as pl
from jax.experimental.pallas import tpu as pltpu
from jax.experimental.pallas import tpu_sc as plsc

NC = 2
NS = 16
LN = 16
K = 120
CW = 24
PAD = 8

_INTERPRET = False


def _encode_body(a_ref, w_ref, al_ref, o_ref):
    a = a_ref[...]
    p = jnp.where(a >= 0.0, a, al_ref[0, 0] * a)
    o_ref[...] = lax.dot_general(
        p, w_ref[...], (((1,), (1,)), ((), ())),
        preferred_element_type=jnp.float32)


def _encode(node_attr, W_enc, prelu_a):
    N, IN = node_attr.shape
    HF = IN // 2
    BN = 1000
    NB = N // BN
    return pl.pallas_call(
        _encode_body,
        grid=(2, NB),
        in_specs=[
            pl.BlockSpec((BN, IN), lambda h, i: (i, 0)),
            pl.BlockSpec((HF, IN), lambda h, i: (h, 0)),
            pl.BlockSpec(memory_space=pltpu.SMEM),
        ],
        out_specs=pl.BlockSpec((BN, HF), lambda h, i: (h * NB + i, 0)),
        out_shape=jax.ShapeDtypeStruct((2 * N, HF), jnp.float32),
        interpret=_INTERPRET,
    )(node_attr, W_enc, prelu_a.reshape(1, 1))


def _sc_aggregate(x_flat, packed, zeros_hbm):
    twoN, HF = x_flat.shape
    N = twoN // 2
    NCHUNK = packed.shape[0]
    CH = NCHUNK // NS
    NPAIR = CH // 2
    NR = N + PAD
    RPT = ((N // NS + 7) // 8) * 8
    RPT_L = N - (NS - 1) * RPT
    NW = N * CW
    NWP = NW + 128
    CPW = ((NW // NS + 127) // 128) * 128
    CPW_L = NW - (NS - 1) * CPW

    mesh = plsc.VectorSubcoreMesh(
        core_axis_name="c", subcore_axis_name="s",
        num_cores=NC, num_subcores=NS)

    def body(x_hbm, idx_hbm, zero_hbm, acc_out, cnt_out,
             ib0, ib1, rows0, rows1, ones_v,
             acc_sh, cnt_sh, gsem, ssem0, ssem1, csem):
        cid = lax.axis_index("c")
        wid = lax.axis_index("s")
        coff = cid * N

        def init_wb(rows, words, store):
            r0 = pl.multiple_of(wid * RPT, 8)
            w0 = pl.multiple_of(wid * CPW, 128)
            if store:
                pltpu.sync_copy(acc_sh.at[pl.ds(r0, rows)],
                                acc_out.at[pl.ds(pl.multiple_of(
                                    coff + wid * RPT, 8), rows)])
                pltpu.sync_copy(
                    cnt_sh.at[pl.ds(w0, words)],
                    cnt_out.at[pl.ds(pl.multiple_of(
                        cid * NW + wid * CPW, 128), words)])
            else:
                pltpu.sync_copy(x_hbm.at[pl.ds(pl.multiple_of(
                                    coff + wid * RPT, 8), rows)],
                                acc_sh.at[pl.ds(r0, rows)])
                pltpu.sync_copy(zero_hbm.at[pl.ds(w0, words)],
                                cnt_sh.at[pl.ds(w0, words)])

        @pl.when(wid < NS - 1)
        def _():
            init_wb(RPT, CPW, False)

        @pl.when(wid == NS - 1)
        def _():
            init_wb(RPT_L, CPW_L, False)
            pltpu.sync_copy(zero_hbm.at[pl.ds(NW, NWP - NW)],
                            cnt_sh.at[pl.ds(NW, NWP - NW)])

        one_offs = list(range(0, K - LN + 1, LN))
        if one_offs[-1] != K - LN:
            one_offs.append(K - LN)
        for o in one_offs:
            ones_v[pl.ds(o, LN)] = jnp.full((LN,), 1.0, jnp.float32)

        plsc.subcore_barrier()

        cbase = wid * CH

        def stage(chunk_idx, ib):
            pltpu.sync_copy(idx_hbm.at[cbase + chunk_idx], ib)

        def wait_scatter(rv, ib, sem):
            pltpu.make_async_copy(rv, acc_sh.at[ib.at[2]], sem).wait()

        def wait_cnt(ib):
            pltpu.make_async_copy(ones_v, cnt_sh.at[ib.at[3]], csem).wait()

        stage(0, ib0)

        def pair(j, carry):
            c0 = 2 * j
            g0 = pltpu.async_copy(x_hbm.at[ib0.at[cid]], rows0, gsem)

            @pl.when(j > 0)
            def _():
                wait_scatter(rows1, ib1, ssem1)

                @pl.when(cid == 1)
                def _():
                    wait_cnt(ib1)
            stage(c0 + 1, ib1)
            g1 = pltpu.async_copy(x_hbm.at[ib1.at[cid]], rows1, gsem)
            g0.wait()
            pltpu.async_copy(rows0, acc_sh.at[ib0.at[2]], ssem0, add=True)

            @pl.when(cid == 0)
            def _():
                pltpu.async_copy(ones_v, cnt_sh.at[ib0.at[3]], csem, add=True)

            @pl.when(j < NPAIR - 1)
            def _():
                wait_scatter(rows0, ib0, ssem0)

                @pl.when(cid == 0)
                def _():
                    wait_cnt(ib0)
                stage(c0 + 2, ib0)
            g1.wait()
            pltpu.async_copy(rows1, acc_sh.at[ib1.at[2]], ssem1, add=True)

            @pl.when(cid == 1)
            def _():
                pltpu.async_copy(ones_v, cnt_sh.at[ib1.at[3]], csem, add=True)
            return carry
        lax.fori_loop(0, NPAIR, pair, 0)

        wait_scatter(rows0, ib0, ssem0)
        wait_scatter(rows1, ib1, ssem1)
        wait_cnt(ib0)

        plsc.subcore_barrier()

        @pl.when(wid < NS - 1)
        def _():
            init_wb(RPT, CPW, True)

        @pl.when(wid == NS - 1)
        def _():
            init_wb(RPT_L, CPW_L, True)

    fn = pl.kernel(
        body,
        out_type=[
            jax.ShapeDtypeStruct((twoN, HF), jnp.float32),
            jax.ShapeDtypeStruct((NC * N * CW,), jnp.float32),
        ],
        mesh=mesh,
        scratch_types=[
            pltpu.VMEM((4, K), jnp.int32),
            pltpu.VMEM((4, K), jnp.int32),
            pltpu.VMEM((K, HF), jnp.float32),
            pltpu.VMEM((K, HF), jnp.float32),
            pltpu.VMEM((K,), jnp.float32),
            pltpu.VMEM_SHARED((NR, HF), jnp.float32),
            pltpu.VMEM_SHARED((NWP,), jnp.float32),
            pltpu.SemaphoreType.DMA,
            pltpu.SemaphoreType.DMA,
            pltpu.SemaphoreType.DMA,
            pltpu.SemaphoreType.DMA,
        ],
        interpret=_INTERPRET,
    )
    return fn(x_flat, packed, zeros_hbm)


def _mlp_body(a0_ref, a1_ref, c0_ref, c1_ref, t_ref, ca_ref, w1_ref, b1_ref,
              w2_ref, b2_ref, o_ref):
    aggr = jnp.concatenate([a0_ref[...], a1_ref[...]], axis=1)
    cnt = c0_ref[...] + c1_ref[...]
    aggr = aggr + lax.dot_general(
        cnt, t_ref[...], (((1,), (0,)), ((), ())),
        preferred_element_type=jnp.float32) + ca_ref[...]
    h = lax.dot_general(
        aggr, w1_ref[...], (((1,), (1,)), ((), ())),
        preferred_element_type=jnp.float32) + b1_ref[...]
    h = jnp.maximum(h, 0.0)
    o_ref[...] = lax.dot_general(
        h, w2_ref[...], (((1,), (1,)), ((), ())),
        preferred_element_type=jnp.float32) + b2_ref[...]


def _mlp(acc, cnt2, T_pad, const_attr, W1, b1, W2, b2):
    twoN, HF = acc.shape
    N = twoN // 2
    IN = 2 * HF
    HID = W1.shape[0]
    OUT = W2.shape[0]
    BN = 1000
    NB = N // BN
    return pl.pallas_call(
        _mlp_body,
        grid=(NB,),
        in_specs=[
            pl.BlockSpec((BN, HF), lambda i: (i, 0)),
            pl.BlockSpec((BN, HF), lambda i: (i + NB, 0)),
            pl.BlockSpec((BN, CW), lambda i: (i, 0)),
            pl.BlockSpec((BN, CW), lambda i: (i + NB, 0)),
            pl.BlockSpec((CW, IN), lambda i: (0, 0)),
            pl.BlockSpec((1, IN), lambda i: (0, 0)),
            pl.BlockSpec((HID, IN), lambda i: (0, 0)),
            pl.BlockSpec((1, HID), lambda i: (0, 0)),
            pl.BlockSpec((OUT, HID), lambda i: (0, 0)),
            pl.BlockSpec((1, OUT), lambda i: (0, 0)),
        ],
        out_specs=pl.BlockSpec((BN, OUT), lambda i: (i, 0)),
        out_shape=jax.ShapeDtypeStruct((N, OUT), jnp.float32),
        interpret=_INTERPRET,
    )(acc, acc, cnt2, cnt2, T_pad, const_attr, W1, b1, W2, b2)


def kernel(node_attr, edge_index, edge_type, edge_dire_type, W_enc, prelu_a,
           emb1, emb2, W1, b1, W2, b2):
    N, IN = node_attr.shape
    E = edge_index.shape[1]
    nd = emb2.shape[0]

    src = edge_index[0].astype(jnp.int32)
    dst = edge_index[1].astype(jnp.int32)
    code = edge_type.astype(jnp.int32) * nd + edge_dire_type.astype(jnp.int32)
    EPK = NS * K * 2
    EP = ((E + EPK - 1) // EPK) * EPK
    npad = EP - E
    src = jnp.concatenate([src, jnp.zeros((npad,), jnp.int32)])
    dst = jnp.concatenate([dst, jnp.full((npad,), N, jnp.int32)])
    code = jnp.concatenate([code, jnp.zeros((npad,), jnp.int32)])
    flat = dst * CW + code
    srcr = src.reshape(-1, K)
    packed = jnp.stack(
        [srcr, srcr + N, dst.reshape(-1, K), flat.reshape(-1, K)], axis=1)
    zeros = jnp.zeros((N * CW + 128,), jnp.float32)

    nb = emb1.shape[0]
    t_idx = jnp.repeat(jnp.arange(nb), nd)
    d_idx = jnp.tile(jnp.arange(nd), nb)
    T = emb1[t_idx] + emb2[d_idx]
    T_pad = jnp.zeros((CW, IN), jnp.float32).at[:nb * nd].set(T)
    const_attr = (emb1[4] + emb2[0]).reshape(1, IN)

    x_flat = _encode(node_attr, W_enc, prelu_a.astype(jnp.float32))
    acc, cnt = _sc_aggregate(x_flat, packed, zeros)
    cnt2 = cnt.reshape(2 * N, CW)
    return _mlp(acc, cnt2, T_pad, const_attr, W1, b1.reshape(1, -1),
                W2, b2.reshape(1, -1))
